# baseline (speedup 1.0000x reference)
import jax
import jax.numpy as jnp
from jax import lax
from jax.experimental import pallas as pl
from jax.experimental.pallas import tpu as pltpu
from jax.experimental.pallas import tpu_sc as plsc

N = 10000
E = 320000
_NC = 2
_NS = 16
_NW = _NC * _NS
_CK = 128
_PASS = 40
_NPASS = 2
_PER_TILE = _CK * _PASS * _NPASS
_EPAD = _NW * _PER_TILE
_NCHUNKS = _EPAD // _CK
_NOUT = 10240
_RPT = _NOUT // _NS
_ZR = 16


def _fill_zero_2d(ref, rows, w):
    def row_body(i, carry):
        def col_body(k, c2):
            ref[i, pl.ds(k * 16, 16)] = jnp.zeros((16,), jnp.float32)
            return c2
        return lax.fori_loop(0, w // 16, col_body, carry)
    lax.fori_loop(0, rows, row_body, 0)


def _fill_const_1d(ref, n, val):
    def body(k, c):
        ref[pl.ds(k * 16, 16)] = jnp.full((16,), val, jnp.float32)
        return c
    lax.fori_loop(0, n // 16, body, 0)


def _make_sc_agg(w, with_deg, spmem_src=False):
    mesh = plsc.VectorSubcoreMesh(core_axis_name="c", subcore_axis_name="s")
    out_type = [jax.ShapeDtypeStruct((_NC, _NOUT, w), jnp.float32)]
    if with_deg:
        out_type.append(jax.ShapeDtypeStruct((_NC, _NOUT), jnp.float32))
    scratch = [
        pltpu.VMEM((_PASS, _CK), jnp.int32),
        pltpu.VMEM((_PASS, _CK), jnp.int32),
        pltpu.VMEM((_CK, w), jnp.float32),
        pltpu.VMEM((_CK, w), jnp.float32),
        pltpu.VMEM((_ZR, w), jnp.float32),
        pltpu.VMEM_SHARED((_NOUT, w), jnp.float32),
        pltpu.SemaphoreType.DMA,
        pltpu.SemaphoreType.DMA,
    ]
    if spmem_src:
        scratch.append(pltpu.VMEM_SHARED((N, w), jnp.float32))
    if with_deg:
        scratch += [
            pltpu.VMEM((_CK,), jnp.float32),
            pltpu.VMEM((2048,), jnp.float32),
            pltpu.VMEM_SHARED((_NOUT,), jnp.float32),
            pltpu.SemaphoreType.DMA,
            pltpu.SemaphoreType.DMA,
        ]

    def body(x_hbm, src_hbm, dst_hbm, *rest):
        if with_deg:
            (part_out, deg_out, src_v, dst_v, buf0, buf1, zbuf,
             accum, sem0, sem1, ones_v, zd, degacc, dsem0, dsem1) = rest
            x_src = x_hbm
        elif spmem_src:
            (part_out, src_v, dst_v, buf0, buf1, zbuf,
             accum, sem0, sem1, xsp) = rest
            x_src = xsp
        else:
            (part_out, src_v, dst_v, buf0, buf1, zbuf,
             accum, sem0, sem1) = rest
            x_src = x_hbm
        c = lax.axis_index("c")
        s = lax.axis_index("s")
        g = c * _NS + s

        _fill_zero_2d(zbuf, _ZR, w)

        def zero_body(r, carry):
            pltpu.sync_copy(zbuf, accum.at[pl.ds(s * _RPT + r * _ZR, _ZR)])
            return carry
        lax.fori_loop(0, _RPT // _ZR, zero_body, 0)

        if spmem_src:
            nrt = N // _NS
            pltpu.sync_copy(x_hbm.at[pl.ds(s * nrt, nrt)],
                            xsp.at[pl.ds(s * nrt, nrt)])

        if with_deg:
            _fill_const_1d(ones_v, _CK, 1.0)
            _fill_const_1d(zd, 2048, 0.0)

            @pl.when(s == 0)
            def _():
                for r in range(5):
                    pltpu.sync_copy(zd, degacc.at[pl.ds(r * 2048, 2048)])

        plsc.subcore_barrier()

        def one_pass(chunk_base):
            pltpu.sync_copy(src_hbm.at[pl.ds(chunk_base, _PASS)], src_v)
            pltpu.sync_copy(dst_hbm.at[pl.ds(chunk_base, _PASS)], dst_v)
            pltpu.async_copy(x_src.at[src_v.at[0]], buf0, sem0)

            def chunk_body(jj, carry):
                j0 = jj * 2
                j1 = j0 + 1
                pltpu.async_copy(x_src.at[src_v.at[j1]], buf1, sem1)
                pltpu.make_async_copy(
                    x_src.at[src_v.at[j0]], buf0, sem0).wait()
                pltpu.sync_copy(buf0, accum.at[dst_v.at[j0]], add=True)
                if with_deg:
                    @pl.when(jj >= 1)
                    def _():
                        pltpu.make_async_copy(
                            ones_v, degacc.at[dst_v.at[0]], dsem0).wait()
                    pltpu.async_copy(ones_v, degacc.at[dst_v.at[j0]],
                                     dsem0, add=True)

                @pl.when(jj < _PASS // 2 - 1)
                def _():
                    pltpu.async_copy(x_src.at[src_v.at[j0 + 2]], buf0, sem0)

                pltpu.make_async_copy(
                    x_src.at[src_v.at[j1]], buf1, sem1).wait()
                pltpu.sync_copy(buf1, accum.at[dst_v.at[j1]], add=True)
                if with_deg:
                    @pl.when(jj >= 1)
                    def _():
                        pltpu.make_async_copy(
                            ones_v, degacc.at[dst_v.at[0]], dsem1).wait()
                    pltpu.async_copy(ones_v, degacc.at[dst_v.at[j1]],
                                     dsem1, add=True)
                return carry

            lax.fori_loop(0, _PASS // 2, chunk_body, 0)
            if with_deg:
                pltpu.make_async_copy(
                    ones_v, degacc.at[dst_v.at[0]], dsem0).wait()
                pltpu.make_async_copy(
                    ones_v, degacc.at[dst_v.at[0]], dsem1).wait()

        for p in range(_NPASS):
            one_pass(g * _PASS * _NPASS + p * _PASS)

        plsc.subcore_barrier()

        pltpu.sync_copy(accum.at[pl.ds(s * _RPT, _RPT)],
                        part_out.at[c, pl.ds(s * _RPT, _RPT)])
        if with_deg:
            @pl.when(s == 0)
            def _():
                pltpu.sync_copy(degacc, deg_out.at[c])

    return pl.kernel(body, mesh=mesh, out_type=out_type,
                     scratch_types=scratch,
                     compiler_params=pltpu.CompilerParams(
                         use_tc_tiling_on_sc=False))


_sc_agg_128 = _make_sc_agg(128, True)
_sc_agg_64 = _make_sc_agg(64, False, spmem_src=True)

_RB = 1024


def _tc0_body(x_ref, ws_ref, b_ref, xs_ref):
    xs_ref[...] = (
        jnp.dot(x_ref[...], ws_ref[...], preferred_element_type=jnp.float32)
        + b_ref[...])


def _tc1_body(xs_ref, parts_ref, deg_ref, wn_ref, wn2_ref, h_ref, p2_ref):
    ssum = parts_ref[0] + parts_ref[1]
    d = deg_ref[0] + deg_ref[1]
    neigh = ssum / jnp.maximum(d, 1.0)[:, None]
    h = xs_ref[...] + jnp.dot(neigh, wn_ref[...],
                              preferred_element_type=jnp.float32)
    h = jnp.maximum(h, 0.0)
    h_ref[...] = h
    p2_ref[...] = jnp.dot(h, wn2_ref[...], preferred_element_type=jnp.float32)


def _tc2_body(hs_ref, parts_ref, deg_ref, out_ref):
    ssum = parts_ref[0] + parts_ref[1]
    d = deg_ref[0] + deg_ref[1]
    out_ref[...] = hs_ref[...] + ssum / jnp.maximum(d, 1.0)[:, None]


def _tc0(x, ws, b):
    return pl.pallas_call(
        _tc0_body,
        grid=(_NOUT // _RB,),
        in_specs=[
            pl.BlockSpec((_RB, 128), lambda i: (i, 0)),
            pl.BlockSpec((128, 128), lambda i: (0, 0)),
            pl.BlockSpec((1, 128), lambda i: (0, 0)),
        ],
        out_specs=pl.BlockSpec((_RB, 128), lambda i: (i, 0)),
        out_shape=jax.ShapeDtypeStruct((N, 128), jnp.float32),
    )(x, ws, b)


def _tc1(xs, parts, deg, wn, wn2):
    return pl.pallas_call(
        _tc1_body,
        grid=(_NOUT // _RB,),
        in_specs=[
            pl.BlockSpec((_RB, 128), lambda i: (i, 0)),
            pl.BlockSpec((_NC, _RB, 128), lambda i: (0, i, 0)),
            pl.BlockSpec((_NC, _RB), lambda i: (0, i)),
            pl.BlockSpec((128, 128), lambda i: (0, 0)),
            pl.BlockSpec((128, 64), lambda i: (0, 0)),
        ],
        out_specs=[
            pl.BlockSpec((_RB, 128), lambda i: (i, 0)),
            pl.BlockSpec((_RB, 64), lambda i: (i, 0)),
        ],
        out_shape=[
            jax.ShapeDtypeStruct((N, 128), jnp.float32),
            jax.ShapeDtypeStruct((N, 64), jnp.float32),
        ],
    )(xs, parts, deg, wn, wn2)


def _tc1b(h, ws, b):
    return pl.pallas_call(
        _tc0_body,
        grid=(_NOUT // _RB,),
        in_specs=[
            pl.BlockSpec((_RB, 128), lambda i: (i, 0)),
            pl.BlockSpec((128, 64), lambda i: (0, 0)),
            pl.BlockSpec((1, 64), lambda i: (0, 0)),
        ],
        out_specs=pl.BlockSpec((_RB, 64), lambda i: (i, 0)),
        out_shape=jax.ShapeDtypeStruct((N, 64), jnp.float32),
    )(h, ws, b)


def _tc2(hs, parts, deg):
    return pl.pallas_call(
        _tc2_body,
        grid=(_NOUT // _RB,),
        in_specs=[
            pl.BlockSpec((_RB, 64), lambda i: (i, 0)),
            pl.BlockSpec((_NC, _RB, 64), lambda i: (0, i, 0)),
            pl.BlockSpec((_NC, _RB), lambda i: (0, i)),
        ],
        out_specs=pl.BlockSpec((_RB, 64), lambda i: (i, 0)),
        out_shape=jax.ShapeDtypeStruct((N, 64), jnp.float32),
    )(hs, parts, deg)


def kernel(features, edge_index, W_neigh1, W_self1, b1, W_neigh2, W_self2,
           b2):
    src = edge_index[0]
    dst = edge_index[1]
    pad = _EPAD - E
    pad_src = jnp.arange(pad, dtype=jnp.int32) % N
    pad_dst = N + jnp.arange(pad, dtype=jnp.int32) % (_NOUT - N)
    src2 = jnp.concatenate([src, pad_src]).reshape(_NCHUNKS, _CK)
    dst2 = jnp.concatenate([dst, pad_dst]).reshape(_NCHUNKS, _CK)

    xs = _tc0(features, W_self1, b1.reshape(1, -1))
    parts1, deg = _sc_agg_128(features, src2, dst2)
    h, p2in = _tc1(xs, parts1, deg, W_neigh1, W_neigh2)
    hs2 = _tc1b(h, W_self2, b2.reshape(1, -1))
    parts2, = _sc_agg_64(p2in, src2, dst2)
    out = _tc2(hs2, parts2, deg)
    return out


# speedup vs baseline: 1.0564x; 1.0564x over previous
import jax
import jax.numpy as jnp
from jax import lax
from jax.experimental import pallas as pl
from jax.experimental.pallas import tpu as pltpu
from jax.experimental.pallas import tpu_sc as plsc

N = 10000
E = 320000
_NC = 2
_NS = 16
_NW = _NC * _NS
_CK = 128
_PASS = 40
_NPASS = 2
_PER_TILE = _CK * _PASS * _NPASS
_EPAD = _NW * _PER_TILE
_NCHUNKS = _EPAD // _CK
_NOUT = 10240
_RPT = _NOUT // _NS
_ZR = 16


def _fill_zero_2d(ref, rows, w):
    def row_body(i, carry):
        def col_body(k, c2):
            ref[i, pl.ds(k * 16, 16)] = jnp.zeros((16,), jnp.float32)
            return c2
        return lax.fori_loop(0, w // 16, col_body, carry)
    lax.fori_loop(0, rows, row_body, 0)


def _fill_const_1d(ref, n, val):
    def body(k, c):
        ref[pl.ds(k * 16, 16)] = jnp.full((16,), val, jnp.float32)
        return c
    lax.fori_loop(0, n // 16, body, 0)


def _make_sc_agg(w, with_deg):
    mesh = plsc.VectorSubcoreMesh(core_axis_name="c", subcore_axis_name="s")
    out_type = [jax.ShapeDtypeStruct((_NC, _NOUT, w), jnp.float32)]
    if with_deg:
        out_type.append(jax.ShapeDtypeStruct((_NC, _NOUT), jnp.float32))
    scratch = [
        pltpu.VMEM((_PASS, _CK), jnp.int32),
        pltpu.VMEM((_PASS, _CK), jnp.int32),
        pltpu.VMEM((_CK, w), jnp.float32),
        pltpu.VMEM((_CK, w), jnp.float32),
        pltpu.VMEM((_ZR, w), jnp.float32),
        pltpu.VMEM_SHARED((_NOUT, w), jnp.float32),
        pltpu.SemaphoreType.DMA,
        pltpu.SemaphoreType.DMA,
    ]
    if with_deg:
        scratch += [
            pltpu.VMEM((_CK,), jnp.float32),
            pltpu.VMEM((2048,), jnp.float32),
            pltpu.VMEM_SHARED((_NOUT,), jnp.float32),
            pltpu.SemaphoreType.DMA,
            pltpu.SemaphoreType.DMA,
        ]

    def body(x_hbm, src_hbm, dst_hbm, *rest):
        if with_deg:
            (part_out, deg_out, src_v, dst_v, buf0, buf1, zbuf,
             accum, sem0, sem1, ones_v, zd, degacc, dsem0, dsem1) = rest
            x_src = x_hbm
        else:
            (part_out, src_v, dst_v, buf0, buf1, zbuf,
             accum, sem0, sem1) = rest
            x_src = x_hbm
        c = lax.axis_index("c")
        s = lax.axis_index("s")
        g = c * _NS + s

        _fill_zero_2d(zbuf, _ZR, w)

        def zero_body(r, carry):
            pltpu.sync_copy(zbuf, accum.at[pl.ds(s * _RPT + r * _ZR, _ZR)])
            return carry
        lax.fori_loop(0, _RPT // _ZR, zero_body, 0)

        if with_deg:
            _fill_const_1d(ones_v, _CK, 1.0)
            _fill_const_1d(zd, 2048, 0.0)

            @pl.when(s == 0)
            def _():
                for r in range(5):
                    pltpu.sync_copy(zd, degacc.at[pl.ds(r * 2048, 2048)])

        plsc.subcore_barrier()

        def one_pass(chunk_base):
            pltpu.sync_copy(src_hbm.at[pl.ds(chunk_base, _PASS)], src_v)
            pltpu.sync_copy(dst_hbm.at[pl.ds(chunk_base, _PASS)], dst_v)
            pltpu.async_copy(x_src.at[src_v.at[0]], buf0, sem0)

            def chunk_body(jj, carry):
                j0 = jj * 2
                j1 = j0 + 1
                pltpu.async_copy(x_src.at[src_v.at[j1]], buf1, sem1)
                pltpu.make_async_copy(
                    x_src.at[src_v.at[j0]], buf0, sem0).wait()
                pltpu.sync_copy(buf0, accum.at[dst_v.at[j0]], add=True)
                if with_deg:
                    @pl.when(jj >= 1)
                    def _():
                        pltpu.make_async_copy(
                            ones_v, degacc.at[dst_v.at[0]], dsem0).wait()
                    pltpu.async_copy(ones_v, degacc.at[dst_v.at[j0]],
                                     dsem0, add=True)

                @pl.when(jj < _PASS // 2 - 1)
                def _():
                    pltpu.async_copy(x_src.at[src_v.at[j0 + 2]], buf0, sem0)

                pltpu.make_async_copy(
                    x_src.at[src_v.at[j1]], buf1, sem1).wait()
                pltpu.sync_copy(buf1, accum.at[dst_v.at[j1]], add=True)
                if with_deg:
                    @pl.when(jj >= 1)
                    def _():
                        pltpu.make_async_copy(
                            ones_v, degacc.at[dst_v.at[0]], dsem1).wait()
                    pltpu.async_copy(ones_v, degacc.at[dst_v.at[j1]],
                                     dsem1, add=True)
                return carry

            lax.fori_loop(0, _PASS // 2, chunk_body, 0)
            if with_deg:
                pltpu.make_async_copy(
                    ones_v, degacc.at[dst_v.at[0]], dsem0).wait()
                pltpu.make_async_copy(
                    ones_v, degacc.at[dst_v.at[0]], dsem1).wait()

        for p in range(_NPASS):
            one_pass(g * _PASS * _NPASS + p * _PASS)

        plsc.subcore_barrier()

        pltpu.sync_copy(accum.at[pl.ds(s * _RPT, _RPT)],
                        part_out.at[c, pl.ds(s * _RPT, _RPT)])
        if with_deg:
            @pl.when(s == 0)
            def _():
                pltpu.sync_copy(degacc, deg_out.at[c])

    return pl.kernel(body, mesh=mesh, out_type=out_type,
                     scratch_types=scratch,
                     compiler_params=pltpu.CompilerParams(
                         use_tc_tiling_on_sc=False))


_sc_agg_128 = _make_sc_agg(128, True)
_sc_agg_64 = _make_sc_agg(64, False)

_RB = 1024


def _tc0_body(x_ref, ws_ref, b_ref, xs_ref):
    xs_ref[...] = (
        jnp.dot(x_ref[...], ws_ref[...], preferred_element_type=jnp.float32)
        + b_ref[...])


def _tc1_body(xs_ref, parts_ref, deg_ref, wn_ref, wn2_ref, h_ref, p2_ref):
    ssum = parts_ref[0] + parts_ref[1]
    d = deg_ref[0] + deg_ref[1]
    neigh = ssum / jnp.maximum(d, 1.0)[:, None]
    h = xs_ref[...] + jnp.dot(neigh, wn_ref[...],
                              preferred_element_type=jnp.float32)
    h = jnp.maximum(h, 0.0)
    h_ref[...] = h
    p2_ref[...] = jnp.dot(h, wn2_ref[...], preferred_element_type=jnp.float32)


def _tc2_body(hs_ref, parts_ref, deg_ref, out_ref):
    ssum = parts_ref[0] + parts_ref[1]
    d = deg_ref[0] + deg_ref[1]
    out_ref[...] = hs_ref[...] + ssum / jnp.maximum(d, 1.0)[:, None]


def _tc0(x, ws, b):
    return pl.pallas_call(
        _tc0_body,
        grid=(_NOUT // _RB,),
        in_specs=[
            pl.BlockSpec((_RB, 128), lambda i: (i, 0)),
            pl.BlockSpec((128, 128), lambda i: (0, 0)),
            pl.BlockSpec((1, 128), lambda i: (0, 0)),
        ],
        out_specs=pl.BlockSpec((_RB, 128), lambda i: (i, 0)),
        out_shape=jax.ShapeDtypeStruct((N, 128), jnp.float32),
    )(x, ws, b)


def _tc1(xs, parts, deg, wn, wn2):
    return pl.pallas_call(
        _tc1_body,
        grid=(_NOUT // _RB,),
        in_specs=[
            pl.BlockSpec((_RB, 128), lambda i: (i, 0)),
            pl.BlockSpec((_NC, _RB, 128), lambda i: (0, i, 0)),
            pl.BlockSpec((_NC, _RB), lambda i: (0, i)),
            pl.BlockSpec((128, 128), lambda i: (0, 0)),
            pl.BlockSpec((128, 64), lambda i: (0, 0)),
        ],
        out_specs=[
            pl.BlockSpec((_RB, 128), lambda i: (i, 0)),
            pl.BlockSpec((_RB, 64), lambda i: (i, 0)),
        ],
        out_shape=[
            jax.ShapeDtypeStruct((N, 128), jnp.float32),
            jax.ShapeDtypeStruct((N, 64), jnp.float32),
        ],
    )(xs, parts, deg, wn, wn2)


def _tc1b(h, ws, b):
    return pl.pallas_call(
        _tc0_body,
        grid=(_NOUT // _RB,),
        in_specs=[
            pl.BlockSpec((_RB, 128), lambda i: (i, 0)),
            pl.BlockSpec((128, 64), lambda i: (0, 0)),
            pl.BlockSpec((1, 64), lambda i: (0, 0)),
        ],
        out_specs=pl.BlockSpec((_RB, 64), lambda i: (i, 0)),
        out_shape=jax.ShapeDtypeStruct((N, 64), jnp.float32),
    )(h, ws, b)


def _tc2(hs, parts, deg):
    return pl.pallas_call(
        _tc2_body,
        grid=(_NOUT // _RB,),
        in_specs=[
            pl.BlockSpec((_RB, 64), lambda i: (i, 0)),
            pl.BlockSpec((_NC, _RB, 64), lambda i: (0, i, 0)),
            pl.BlockSpec((_NC, _RB), lambda i: (0, i)),
        ],
        out_specs=pl.BlockSpec((_RB, 64), lambda i: (i, 0)),
        out_shape=jax.ShapeDtypeStruct((N, 64), jnp.float32),
    )(hs, parts, deg)


def kernel(features, edge_index, W_neigh1, W_self1, b1, W_neigh2, W_self2,
           b2):
    src = edge_index[0]
    dst = edge_index[1]
    pad = _EPAD - E
    pad_src = jnp.arange(pad, dtype=jnp.int32) % N
    pad_dst = N + jnp.arange(pad, dtype=jnp.int32) % (_NOUT - N)
    src2 = jnp.concatenate([src, pad_src]).reshape(_NCHUNKS, _CK)
    dst2 = jnp.concatenate([dst, pad_dst]).reshape(_NCHUNKS, _CK)

    xs = _tc0(features, W_self1, b1.reshape(1, -1))
    parts1, deg = _sc_agg_128(features, src2, dst2)
    h, p2in = _tc1(xs, parts1, deg, W_neigh1, W_neigh2)
    hs2 = _tc1b(h, W_self2, b2.reshape(1, -1))
    parts2, = _sc_agg_64(p2in, src2, dst2)
    out = _tc2(hs2, parts2, deg)
    return out
